# Initial kernel scaffold; baseline (speedup 1.0000x reference)
import jax
import jax.numpy as jnp
from jax.experimental import pallas as pl


def kernel(x, pretrained_table, trainable_table):
    raise NotImplementedError("write your pallas kernel here")


# speedup vs baseline: 2.7549x; 2.7549x over previous
import functools

import jax
import jax.numpy as jnp
from jax import lax
from jax.experimental import pallas as pl
from jax.experimental.pallas import tpu as pltpu
from jax.experimental.pallas import tpu_sc as plsc

_NUM_SPEC = 100
_LANES = 16


def _build(N, V, E, NC, NS):
    NW = NC * NS
    n_per_w = N // NW
    CHUNK = 512
    SUB = 128
    n_chunks = n_per_w // CHUNK
    mesh = plsc.VectorSubcoreMesh(
        core_axis_name="c", subcore_axis_name="s",
        num_cores=NC, num_subcores=NS)

    @functools.partial(
        pl.kernel,
        out_type=jax.ShapeDtypeStruct((N, E), jnp.float32),
        mesh=mesh,
        scratch_types=[
            pltpu.VMEM((CHUNK,), jnp.int32),
            pltpu.VMEM((CHUNK, E), jnp.float32),
            pltpu.VMEM((_NUM_SPEC, E), jnp.float32),
            pltpu.SemaphoreType.DMA,
        ],
        compiler_params=pltpu.CompilerParams(use_tc_tiling_on_sc=False),
    )
    def body(x_hbm, p_hbm, t_hbm, out_hbm, idx_v, rows_v, tt_v, sem):
        wid = lax.axis_index("s") * NC + lax.axis_index("c")
        base = wid * n_per_w
        pltpu.sync_copy(t_hbm, tt_v)

        def chunk_body(g, carry):
            pos = base + g * CHUNK
            pltpu.sync_copy(x_hbm.at[pl.ds(pos, CHUNK)], idx_v)
            for k in range(CHUNK // SUB):
                pltpu.async_copy(
                    p_hbm.at[idx_v.at[pl.ds(k * SUB, SUB)]],
                    rows_v.at[pl.ds(k * SUB, SUB)],
                    sem,
                ).wait()

            def spec_body(j, acc):
                v = idx_v[pl.ds(j * _LANES, _LANES)]
                return acc | ((v - _NUM_SPEC) >> 31)

            accm = lax.fori_loop(
                0, CHUNK // _LANES, spec_body,
                jnp.zeros((_LANES,), jnp.int32))
            any_spec = accm[0]
            for lane in range(1, _LANES):
                any_spec = any_spec | accm[lane]

            @pl.when(any_spec != 0)
            def _fixup():
                def fix_body(j, carry2):
                    xv = idx_v[pl.ds(j * _LANES, _LANES)]
                    for p in range(_LANES):
                        xi = xv[p]
                        is_spec = xi < _NUM_SPEC
                        xs = jnp.where(is_spec, xi, 0)
                        m = jnp.where(is_spec, 1.0, 0.0).astype(jnp.float32)
                        row = j * _LANES + p
                        for h in range(E // _LANES):
                            sl = pl.ds(h * _LANES, _LANES)
                            th = tt_v[xs, sl]
                            ph = rows_v[row, sl]
                            rows_v[row, sl] = ph + m * (th - ph)
                    return carry2

                lax.fori_loop(0, CHUNK // _LANES, fix_body, 0)

            pltpu.sync_copy(rows_v, out_hbm.at[pl.ds(pos, CHUNK)])
            return carry

        lax.fori_loop(0, n_chunks, chunk_body, 0)

    return body


def kernel(x, pretrained_table, trainable_table):
    B, S = x.shape
    V, E = pretrained_table.shape
    N = B * S
    info = plsc.get_sparse_core_info()
    NC, NS = info.num_cores, info.num_subcores
    fn = _build(N, V, E, NC, NS)
    out = fn(x.reshape(N), pretrained_table, trainable_table)
    return out.reshape(B, S, E)


# speedup vs baseline: 3.0694x; 1.1142x over previous
import functools

import jax
import jax.numpy as jnp
from jax import lax
from jax.experimental import pallas as pl
from jax.experimental.pallas import tpu as pltpu
from jax.experimental.pallas import tpu_sc as plsc

_NUM_SPEC = 100
_LANES = 16
_NSLOT = 2


def _build(N, V, E, NC, NS):
    NW = NC * NS
    n_per_w = N // NW
    CHUNK = 1280
    SUB = 128
    n_chunks = n_per_w // CHUNK
    mesh = plsc.VectorSubcoreMesh(
        core_axis_name="c", subcore_axis_name="s",
        num_cores=NC, num_subcores=NS)

    @functools.partial(
        pl.kernel,
        out_type=jax.ShapeDtypeStruct((N, E), jnp.float32),
        mesh=mesh,
        scratch_types=[
            pltpu.VMEM((_NSLOT, CHUNK), jnp.int32),
            pltpu.VMEM((_NSLOT, CHUNK, E), jnp.float32),
            pltpu.VMEM((_NUM_SPEC, E), jnp.float32),
            pltpu.SemaphoreType.DMA,
            pltpu.SemaphoreType.DMA,
            pltpu.SemaphoreType.DMA,
            pltpu.SemaphoreType.DMA,
            pltpu.SemaphoreType.DMA,
            pltpu.SemaphoreType.DMA,
        ],
        compiler_params=pltpu.CompilerParams(use_tc_tiling_on_sc=False),
    )
    def body(x_hbm, p_hbm, t_hbm, out_hbm, idx_v, rows_v, tt_v,
             si0, si1, sg0, sg1, so0, so1):
        sis = (si0, si1)
        sgs = (sg0, sg1)
        sos = (so0, so1)
        wid = lax.axis_index("s") * NC + lax.axis_index("c")
        base = wid * n_per_w
        pltpu.sync_copy(t_hbm, tt_v)

        def idx_copy(s, g):
            return pltpu.make_async_copy(
                x_hbm.at[pl.ds(base + g * CHUNK, CHUNK)], idx_v.at[s], sis[s])

        def gather_copy(s, k):
            return pltpu.make_async_copy(
                p_hbm.at[idx_v.at[s, pl.ds(k * SUB, SUB)]],
                rows_v.at[s, pl.ds(k * SUB, SUB)], sgs[s])

        def out_copy(s, g):
            return pltpu.make_async_copy(
                rows_v.at[s], out_hbm.at[pl.ds(base + g * CHUNK, CHUNK)],
                sos[s])

        def start_gather(s):
            for k in range(CHUNK // SUB):
                gather_copy(s, k).start()

        def wait_gather(s):
            for k in range(CHUNK // SUB):
                gather_copy(s, k).wait()

        def process(s):
            def spec_body(j, acc):
                v = idx_v[s, pl.ds(j * _LANES, _LANES)]
                return acc | ((v - _NUM_SPEC) >> 31)

            accm = lax.fori_loop(
                0, CHUNK // _LANES, spec_body,
                jnp.zeros((_LANES,), jnp.int32))
            any_spec = accm[0]
            for lane in range(1, _LANES):
                any_spec = any_spec | accm[lane]

            @pl.when(any_spec != 0)
            def _fixup():
                def fix_body(j, carry2):
                    xv = idx_v[s, pl.ds(j * _LANES, _LANES)]
                    for p in range(_LANES):
                        xi = xv[p]
                        is_spec = xi < _NUM_SPEC
                        xs = jnp.where(is_spec, xi, 0)
                        m = jnp.where(is_spec, 1.0, 0.0).astype(jnp.float32)
                        row = j * _LANES + p
                        for h in range(E // _LANES):
                            sl = pl.ds(h * _LANES, _LANES)
                            th = tt_v[xs, sl]
                            ph = rows_v[s, row, sl]
                            rows_v[s, row, sl] = ph + m * (th - ph)
                    return carry2

                lax.fori_loop(0, CHUNK // _LANES, fix_body, 0)

        idx_copy(0, 0).start()
        idx_copy(0, 0).wait()
        start_gather(0)
        idx_copy(1, 1).start()

        def pair_body(i, carry):
            for b in range(_NSLOT):
                g = i * _NSLOT + b
                s = b
                s2 = (b + 1) % _NSLOT

                @pl.when(g + 1 < n_chunks)
                def _start_next():
                    idx_copy(s2, g + 1).wait()

                    @pl.when(g >= 1)
                    def _drain_prev_out():
                        out_copy(s2, g - 1).wait()

                    start_gather(s2)

                wait_gather(s)
                process(s)
                out_copy(s, g).start()

                @pl.when(g + 2 < n_chunks)
                def _prefetch_idx():
                    idx_copy(s, g + 2).start()

            return carry

        lax.fori_loop(0, n_chunks // _NSLOT, pair_body, 0)
        out_copy(0, n_chunks - 2).wait()
        out_copy(1, n_chunks - 1).wait()

    return body


def kernel(x, pretrained_table, trainable_table):
    B, S = x.shape
    V, E = pretrained_table.shape
    N = B * S
    info = plsc.get_sparse_core_info()
    NC, NS = info.num_cores, info.num_subcores
    fn = _build(N, V, E, NC, NS)
    out = fn(x.reshape(N), pretrained_table, trainable_table)
    return out.reshape(B, S, E)


# speedup vs baseline: 3.2149x; 1.0474x over previous
import functools

import jax
import jax.numpy as jnp
from jax import lax
from jax.experimental import pallas as pl
from jax.experimental.pallas import tpu as pltpu
from jax.experimental.pallas import tpu_sc as plsc

_NUM_SPEC = 100
_LANES = 16
_NSLOT = 2


def _build(N, V, E, NC, NS):
    NW = NC * NS
    n_per_w = N // NW
    CHUNK = 1280
    SUB = 128
    n_chunks = n_per_w // CHUNK
    mesh = plsc.VectorSubcoreMesh(
        core_axis_name="c", subcore_axis_name="s",
        num_cores=NC, num_subcores=NS)

    @functools.partial(
        pl.kernel,
        out_type=jax.ShapeDtypeStruct((N, E), jnp.float32),
        mesh=mesh,
        scratch_types=[
            pltpu.VMEM((_NSLOT, CHUNK), jnp.int32),
            pltpu.VMEM((_NSLOT, CHUNK, E), jnp.float32),
            pltpu.VMEM((_NUM_SPEC, E), jnp.float32),
            pltpu.SemaphoreType.DMA,
            pltpu.SemaphoreType.DMA,
            pltpu.SemaphoreType.DMA,
            pltpu.SemaphoreType.DMA,
            pltpu.SemaphoreType.DMA,
            pltpu.SemaphoreType.DMA,
        ],
        compiler_params=pltpu.CompilerParams(use_tc_tiling_on_sc=False),
    )
    def body(x_hbm, p_hbm, t_hbm, out_hbm, idx_v, rows_v, tt_v,
             si0, si1, sg0, sg1, so0, so1):
        sis = (si0, si1)
        sgs = (sg0, sg1)
        sos = (so0, so1)
        wid = lax.axis_index("s") * NC + lax.axis_index("c")
        base = wid * n_per_w
        pltpu.sync_copy(t_hbm, tt_v)

        def idx_copy(s, g):
            return pltpu.make_async_copy(
                x_hbm.at[pl.ds(base + g * CHUNK, CHUNK)], idx_v.at[s], sis[s])

        def gather_copy(s, k):
            return pltpu.make_async_copy(
                p_hbm.at[idx_v.at[s, pl.ds(k * SUB, SUB)]],
                rows_v.at[s, pl.ds(k * SUB, SUB)], sgs[s])

        def out_copy(s, g):
            return pltpu.make_async_copy(
                rows_v.at[s], out_hbm.at[pl.ds(base + g * CHUNK, CHUNK)],
                sos[s])

        def start_gather(s):
            for k in range(CHUNK // SUB):
                gather_copy(s, k).start()

        def wait_gather(s):
            for k in range(CHUNK // SUB):
                gather_copy(s, k).wait()

        def process(s):
            def spec_body(j, acc):
                v = idx_v[s, pl.ds(j * _LANES, _LANES)]
                return acc | ((v - _NUM_SPEC) >> 31)

            accm = lax.fori_loop(
                0, CHUNK // _LANES, spec_body,
                jnp.zeros((_LANES,), jnp.int32))
            any_spec = accm[0]
            for lane in range(1, _LANES):
                any_spec = any_spec | accm[lane]

            @pl.when(any_spec != 0)
            def _fixup():
                def fix_body(j, carry2):
                    xv = idx_v[s, pl.ds(j * _LANES, _LANES)]
                    for p in range(_LANES):
                        xi = xv[p]
                        is_spec = xi < _NUM_SPEC
                        xs = jnp.where(is_spec, xi, 0)
                        m = jnp.where(is_spec, 1.0, 0.0).astype(jnp.float32)
                        row = j * _LANES + p
                        for h in range(E // _LANES):
                            sl = pl.ds(h * _LANES, _LANES)
                            th = tt_v[xs, sl]
                            ph = rows_v[s, row, sl]
                            rows_v[s, row, sl] = ph + m * (th - ph)
                    return carry2

                lax.fori_loop(0, CHUNK // _LANES, fix_body, 0)

        idx_copy(0, 0).start()
        idx_copy(0, 0).wait()
        start_gather(0)
        idx_copy(1, 1).start()

        def pair_body(i, carry):
            for b in range(_NSLOT):
                g = i * _NSLOT + b
                s = b
                s2 = (b + 1) % _NSLOT

                @pl.when(g + 1 < n_chunks)
                def _start_next():
                    idx_copy(s2, g + 1).wait()

                    @pl.when(g >= 1)
                    def _drain_prev_out():
                        out_copy(s2, g - 1).wait()

                    start_gather(s2)

                wait_gather(s)
                process(s)
                out_copy(s, g).start()

                @pl.when(g + 2 < n_chunks)
                def _prefetch_idx():
                    idx_copy(s, g + 2).start()

            return carry

        lax.fori_loop(0, n_chunks // _NSLOT, pair_body, 0)
        out_copy(0, n_chunks - 2).wait()
        out_copy(1, n_chunks - 1).wait()

    return body


def kernel(x, pretrained_table, trainable_table):
    B, S = x.shape
    V, E = pretrained_table.shape
    N = B * S
    info = plsc.get_sparse_core_info()
    NC, NS = info.num_cores, info.num_subcores
    fn = _build(N, V, E, NC, NS)
    out = fn(x.T.reshape(N), pretrained_table, trainable_table)
    return out.reshape(S, B, E).transpose(1, 0, 2)
